# Initial kernel scaffold; baseline (speedup 1.0000x reference)
import functools

import jax
import jax.numpy as jnp
from jax import lax
from jax.experimental import pallas as pl
from jax.experimental.pallas import tpu as pltpu
from jax.experimental.pallas import tpu_sc as plsc

K, R, KAPPA, NX, NF, DOUT = 4, 16384, 32, 3, 13, 16
NIN = NX + NF
NPTS = K * R
NIDX = NPTS * KAPPA

NC, NS = 2, 16
NW = NC * NS
PTS_PER_W = NPTS // NW

IDX_PER_DMA = 128
PTS_PER_DMA = IDX_PER_DMA // KAPPA
DMAS_PER_BUF = 16
PTS_PER_BUF = DMAS_PER_BUF * PTS_PER_DMA
BUFS_PER_W = PTS_PER_W // PTS_PER_BUF
IDX_ROWS_TOTAL = NIDX // IDX_PER_DMA
IDX_ROWS_PER_W = PTS_PER_W * KAPPA // IDX_PER_DMA

ROWS_BLK = 4096


def _mlp_body(p_ref, h_ref, z_ref):
    z_ref[...] = jnp.maximum(
        jnp.dot(p_ref[...], h_ref[...], preferred_element_type=jnp.float32), 0.0)


def _dense_body(p_ref, g_ref, b_ref, y_ref):
    y_ref[...] = (
        jnp.dot(p_ref[...], g_ref[...], preferred_element_type=jnp.float32)
        + b_ref[...])


_sc_mesh = plsc.VectorSubcoreMesh(core_axis_name="c", subcore_axis_name="s")


@functools.partial(
    pl.kernel,
    out_type=jax.ShapeDtypeStruct((NPTS, DOUT), jnp.float32),
    mesh=_sc_mesh,
    scratch_types=[
        pltpu.VMEM((DMAS_PER_BUF, IDX_PER_DMA), jnp.int32),
        pltpu.VMEM((PTS_PER_BUF * KAPPA, DOUT), jnp.float32),
        pltpu.VMEM((PTS_PER_BUF, DOUT), jnp.float32),
        pltpu.SemaphoreType.DMA,
    ],
)
def _gather_max(z_hbm, nidx_hbm, out_hbm, idx_v, rows_v, out_v, sem):
    wid = lax.axis_index("s") * NC + lax.axis_index("c")
    idx_row0 = wid * IDX_ROWS_PER_W
    pt0 = wid * PTS_PER_W

    def buf_body(t, carry):
        pltpu.sync_copy(
            nidx_hbm.at[pl.ds(idx_row0 + t * DMAS_PER_BUF, DMAS_PER_BUF)],
            idx_v)
        cps = [
            pltpu.async_copy(
                z_hbm.at[idx_v.at[j]],
                rows_v.at[pl.ds(j * IDX_PER_DMA, IDX_PER_DMA)],
                sem)
            for j in range(DMAS_PER_BUF)
        ]
        for cp in cps:
            cp.wait()

        def pt_body(p, c):
            base = p * KAPPA
            acc = rows_v[base]
            for i in range(1, KAPPA):
                acc = jnp.maximum(acc, rows_v[base + i])
            out_v[p] = acc
            return c

        lax.fori_loop(0, PTS_PER_BUF, pt_body, 0, unroll=2)
        pltpu.sync_copy(
            out_v, out_hbm.at[pl.ds(pt0 + t * PTS_PER_BUF, PTS_PER_BUF)])
        return carry

    lax.fori_loop(0, BUFS_PER_W, buf_body, 0)


def kernel(X, F, N, H, Gamma, gamma_bias):
    pflat = jnp.concatenate([X, F], axis=2).reshape(NPTS, NIN)
    nflat = (N + (jnp.arange(K, dtype=jnp.int32) * R)[:, None, None]).reshape(
        IDX_ROWS_TOTAL, IDX_PER_DMA)

    z = pl.pallas_call(
        _mlp_body,
        grid=(NPTS // ROWS_BLK,),
        in_specs=[
            pl.BlockSpec((ROWS_BLK, NIN), lambda i: (i, 0)),
            pl.BlockSpec((NIN, DOUT), lambda i: (0, 0)),
        ],
        out_specs=pl.BlockSpec((ROWS_BLK, DOUT), lambda i: (i, 0)),
        out_shape=jax.ShapeDtypeStruct((NPTS, DOUT), jnp.float32),
    )(pflat, H.T)

    y0 = _gather_max(z, nflat)

    y = pl.pallas_call(
        _dense_body,
        grid=(NPTS // ROWS_BLK,),
        in_specs=[
            pl.BlockSpec((ROWS_BLK, DOUT), lambda i: (i, 0)),
            pl.BlockSpec((DOUT, DOUT), lambda i: (0, 0)),
            pl.BlockSpec((1, DOUT), lambda i: (0, 0)),
        ],
        out_specs=pl.BlockSpec((ROWS_BLK, DOUT), lambda i: (i, 0)),
        out_shape=jax.ShapeDtypeStruct((NPTS, DOUT), jnp.float32),
    )(y0, Gamma, gamma_bias[None, :])

    return y.reshape(K, R, DOUT)


# speedup vs baseline: 100.4498x; 100.4498x over previous
import functools

import jax
import jax.numpy as jnp
from jax import lax
from jax.experimental import pallas as pl
from jax.experimental.pallas import tpu as pltpu
from jax.experimental.pallas import tpu_sc as plsc

K, R, KAPPA, NX, NF, DOUT = 4, 16384, 32, 3, 13, 16
NIN = NX + NF
NPTS = K * R
NIDX = NPTS * KAPPA

NC, NS = 2, 16
NW = NC * NS
PTS_PER_W = NPTS // NW

IDX_PER_DMA = 128
PTS_PER_DMA = IDX_PER_DMA // KAPPA
DMAS_PER_BUF = 16
PTS_PER_BUF = DMAS_PER_BUF * PTS_PER_DMA
BUFS_PER_W = PTS_PER_W // PTS_PER_BUF
IDX_ROWS_TOTAL = NIDX // IDX_PER_DMA
IDX_ROWS_PER_W = PTS_PER_W * KAPPA // IDX_PER_DMA

ROWS_BLK = 4096


def _mlp_body(p_ref, h_ref, z_ref):
    z_ref[...] = jnp.maximum(
        jnp.dot(p_ref[...], h_ref[...], preferred_element_type=jnp.float32), 0.0)


def _dense_body(p_ref, g_ref, b_ref, y_ref):
    y_ref[...] = (
        jnp.dot(p_ref[...], g_ref[...], preferred_element_type=jnp.float32)
        + b_ref[...])


_sc_mesh = plsc.VectorSubcoreMesh(core_axis_name="c", subcore_axis_name="s")


@functools.partial(
    pl.kernel,
    out_type=jax.ShapeDtypeStruct((NPTS, DOUT), jnp.float32),
    mesh=_sc_mesh,
    compiler_params=pltpu.CompilerParams(use_tc_tiling_on_sc=False),
    scratch_types=[
        pltpu.VMEM((DMAS_PER_BUF, IDX_PER_DMA), jnp.int32),
        pltpu.VMEM((PTS_PER_BUF * KAPPA, DOUT), jnp.float32),
        pltpu.VMEM((PTS_PER_BUF, DOUT), jnp.float32),
        pltpu.SemaphoreType.DMA,
    ],
)
def _gather_max(z_hbm, nidx_hbm, out_hbm, idx_v, rows_v, out_v, sem):
    wid = lax.axis_index("s") * NC + lax.axis_index("c")
    idx_row0 = wid * IDX_ROWS_PER_W
    pt0 = wid * PTS_PER_W

    def buf_body(t, carry):
        pltpu.sync_copy(
            nidx_hbm.at[pl.ds(idx_row0 + t * DMAS_PER_BUF, DMAS_PER_BUF)],
            idx_v)
        cps = [
            pltpu.async_copy(
                z_hbm.at[idx_v.at[j]],
                rows_v.at[pl.ds(j * IDX_PER_DMA, IDX_PER_DMA)],
                sem)
            for j in range(DMAS_PER_BUF)
        ]
        for cp in cps:
            cp.wait()

        def pt_body(p, c):
            base = p * KAPPA
            acc = rows_v[base]
            for i in range(1, KAPPA):
                acc = jnp.maximum(acc, rows_v[base + i])
            out_v[p] = acc
            return c

        lax.fori_loop(0, PTS_PER_BUF, pt_body, 0, unroll=2)
        pltpu.sync_copy(
            out_v, out_hbm.at[pl.ds(pt0 + t * PTS_PER_BUF, PTS_PER_BUF)])
        return carry

    lax.fori_loop(0, BUFS_PER_W, buf_body, 0)


def kernel(X, F, N, H, Gamma, gamma_bias):
    pflat = jnp.concatenate([X, F], axis=2).reshape(NPTS, NIN)
    nflat = (N + (jnp.arange(K, dtype=jnp.int32) * R)[:, None, None]).reshape(
        IDX_ROWS_TOTAL, IDX_PER_DMA)

    z = pl.pallas_call(
        _mlp_body,
        grid=(NPTS // ROWS_BLK,),
        in_specs=[
            pl.BlockSpec((ROWS_BLK, NIN), lambda i: (i, 0)),
            pl.BlockSpec((NIN, DOUT), lambda i: (0, 0)),
        ],
        out_specs=pl.BlockSpec((ROWS_BLK, DOUT), lambda i: (i, 0)),
        out_shape=jax.ShapeDtypeStruct((NPTS, DOUT), jnp.float32),
    )(pflat, H.T)

    y0 = _gather_max(z, nflat)

    y = pl.pallas_call(
        _dense_body,
        grid=(NPTS // ROWS_BLK,),
        in_specs=[
            pl.BlockSpec((ROWS_BLK, DOUT), lambda i: (i, 0)),
            pl.BlockSpec((DOUT, DOUT), lambda i: (0, 0)),
            pl.BlockSpec((1, DOUT), lambda i: (0, 0)),
        ],
        out_specs=pl.BlockSpec((ROWS_BLK, DOUT), lambda i: (i, 0)),
        out_shape=jax.ShapeDtypeStruct((NPTS, DOUT), jnp.float32),
    )(y0, Gamma, gamma_bias[None, :])

    return y.reshape(K, R, DOUT)


# speedup vs baseline: 127.9889x; 1.2742x over previous
import functools

import jax
import jax.numpy as jnp
from jax import lax
from jax.experimental import pallas as pl
from jax.experimental.pallas import tpu as pltpu
from jax.experimental.pallas import tpu_sc as plsc

K, R, KAPPA, NX, NF, DOUT = 4, 16384, 32, 3, 13, 16
NIN = NX + NF
NPTS = K * R
NIDX = NPTS * KAPPA

NC, NS = 2, 16
NW = NC * NS
PTS_PER_W = NPTS // NW

IDX_PER_DMA = 128
PTS_PER_DMA = IDX_PER_DMA // KAPPA
DMAS_PER_BUF = 16
PTS_PER_BUF = DMAS_PER_BUF * PTS_PER_DMA
BUFS_PER_W = PTS_PER_W // PTS_PER_BUF
IDX_ROWS_TOTAL = NIDX // IDX_PER_DMA
IDX_ROWS_PER_W = PTS_PER_W * KAPPA // IDX_PER_DMA

ROWS_BLK = 4096


def _mlp_body(p_ref, h_ref, z_ref):
    z_ref[...] = jnp.maximum(
        jnp.dot(p_ref[...], h_ref[...], preferred_element_type=jnp.float32), 0.0)


def _dense_body(p_ref, g_ref, b_ref, y_ref):
    y_ref[...] = (
        jnp.dot(p_ref[...], g_ref[...], preferred_element_type=jnp.float32)
        + b_ref[...])


_sc_mesh = plsc.VectorSubcoreMesh(core_axis_name="c", subcore_axis_name="s")


HALF_BUFS = BUFS_PER_W // 2


@functools.partial(
    pl.kernel,
    out_type=jax.ShapeDtypeStruct((NPTS, DOUT), jnp.float32),
    mesh=_sc_mesh,
    compiler_params=pltpu.CompilerParams(use_tc_tiling_on_sc=False),
    scratch_types=[
        pltpu.VMEM((2, DMAS_PER_BUF, IDX_PER_DMA), jnp.int32),
        pltpu.VMEM((2, PTS_PER_BUF * KAPPA, DOUT), jnp.float32),
        pltpu.VMEM((2, PTS_PER_BUF, DOUT), jnp.float32),
        pltpu.SemaphoreType.DMA,
        pltpu.SemaphoreType.DMA,
    ],
)
def _gather_max(z_hbm, nidx_hbm, out_hbm, idx_v, rows_v, out_v, sem0, sem1):
    wid = lax.axis_index("s") * NC + lax.axis_index("c")
    idx_row0 = wid * IDX_ROWS_PER_W
    pt0 = wid * PTS_PER_W
    sems = (sem0, sem1)

    def fire(parity, t):
        pltpu.sync_copy(
            nidx_hbm.at[pl.ds(idx_row0 + t * DMAS_PER_BUF, DMAS_PER_BUF)],
            idx_v.at[parity])
        for j in range(DMAS_PER_BUF):
            pltpu.async_copy(
                z_hbm.at[idx_v.at[parity, j]],
                rows_v.at[parity, pl.ds(j * IDX_PER_DMA, IDX_PER_DMA)],
                sems[parity])

    def drain(parity):
        pltpu.make_async_copy(
            z_hbm.at[pl.ds(0, PTS_PER_BUF * KAPPA)],
            rows_v.at[parity], sems[parity]).wait()

    def compute(parity, t):
        def pt_body(p, c):
            base = p * KAPPA
            acc = rows_v[parity, base]
            for i in range(1, KAPPA):
                acc = jnp.maximum(acc, rows_v[parity, base + i])
            out_v[parity, p] = acc
            return c

        lax.fori_loop(0, PTS_PER_BUF, pt_body, 0, unroll=2)
        pltpu.sync_copy(
            out_v.at[parity],
            out_hbm.at[pl.ds(pt0 + t * PTS_PER_BUF, PTS_PER_BUF)])

    fire(0, 0)
    fire(1, 1)

    def buf_body(t2, carry):
        drain(0)
        compute(0, 2 * t2)

        @pl.when(t2 < HALF_BUFS - 1)
        def _():
            fire(0, 2 * t2 + 2)

        drain(1)
        compute(1, 2 * t2 + 1)

        @pl.when(t2 < HALF_BUFS - 1)
        def _():
            fire(1, 2 * t2 + 3)

        return carry

    lax.fori_loop(0, HALF_BUFS, buf_body, 0)


def kernel(X, F, N, H, Gamma, gamma_bias):
    pflat = jnp.concatenate([X, F], axis=2).reshape(NPTS, NIN)
    nflat = (N + (jnp.arange(K, dtype=jnp.int32) * R)[:, None, None]).reshape(
        IDX_ROWS_TOTAL, IDX_PER_DMA)

    z = pl.pallas_call(
        _mlp_body,
        grid=(NPTS // ROWS_BLK,),
        in_specs=[
            pl.BlockSpec((ROWS_BLK, NIN), lambda i: (i, 0)),
            pl.BlockSpec((NIN, DOUT), lambda i: (0, 0)),
        ],
        out_specs=pl.BlockSpec((ROWS_BLK, DOUT), lambda i: (i, 0)),
        out_shape=jax.ShapeDtypeStruct((NPTS, DOUT), jnp.float32),
    )(pflat, H.T)

    y0 = _gather_max(z, nflat)

    y = pl.pallas_call(
        _dense_body,
        grid=(NPTS // ROWS_BLK,),
        in_specs=[
            pl.BlockSpec((ROWS_BLK, DOUT), lambda i: (i, 0)),
            pl.BlockSpec((DOUT, DOUT), lambda i: (0, 0)),
            pl.BlockSpec((1, DOUT), lambda i: (0, 0)),
        ],
        out_specs=pl.BlockSpec((ROWS_BLK, DOUT), lambda i: (i, 0)),
        out_shape=jax.ShapeDtypeStruct((NPTS, DOUT), jnp.float32),
    )(y0, Gamma, gamma_bias[None, :])

    return y.reshape(K, R, DOUT)
